# baseline (speedup 1.0000x reference)
import functools

import jax
import jax.numpy as jnp
from jax import lax
from jax.experimental import pallas as pl
from jax.experimental.pallas import tpu as pltpu
from jax.experimental.pallas import tpu_sc as plsc

_EPS = 1e-6
_TILE_N = 512
_MASKED = float(jnp.finfo(jnp.float32).min)
_CD11 = (((1,), (1,)), ((), ()))
_CD10 = (((1,), (0,)), ((), ()))


def _proj_body(q_ref, wqd_ref, bqd_ref, wqu_ref, bqu_ref, wh_ref, bh_ref,
               qnw_ref, qh_ref, hw_ref):
    q = q_ref[...]
    ql = lax.dot_general(q, wqd_ref[...], _CD11,
                         preferred_element_type=jnp.float32) + bqd_ref[...]
    ql = ql * lax.rsqrt(jnp.mean(ql * ql, axis=-1, keepdims=True) + _EPS)
    ql = ql * qnw_ref[...]
    qh_ref[...] = lax.dot_general(ql, wqu_ref[...], _CD11,
                                  preferred_element_type=jnp.float32) + bqu_ref[...]
    logits = lax.dot_general(q, wh_ref[...], _CD11,
                             preferred_element_type=jnp.float32) + bh_ref[...]
    m = jnp.max(logits, axis=-1, keepdims=True)
    e = jnp.exp(logits - m)
    hw_ref[...] = e / jnp.sum(e, axis=-1, keepdims=True)


def _score_body(qh_ref, hw_ref, wk_ref, bk_ref, knw_ref, blocks_ref, mask_ref,
                out_ref):
    x = blocks_ref[0]
    kr = lax.dot_general(wk_ref[...], x, _CD11,
                         preferred_element_type=jnp.float32)
    krb = kr + bk_ref[...]
    sq = krb * krb
    part = sq[0:8, :]
    for i in range(1, 8):
        part = part + sq[8 * i:8 * i + 8, :]
    part = part[0:4, :] + part[4:8, :]
    part = part[0:2, :] + part[2:4, :]
    ms = (part[0:1, :] + part[1:2, :]) * (1.0 / 64.0)
    rs = lax.rsqrt(ms + _EPS)
    keys = (krb * rs) * knw_ref[...]
    sh = lax.dot_general(qh_ref[0].astype(jnp.bfloat16),
                         keys.astype(jnp.bfloat16), _CD10,
                         preferred_element_type=jnp.float32)
    prod = jnp.maximum(sh, 0.0) * hw_ref[0]
    acc = prod[0:8, :] + prod[8:16, :]
    acc = acc[0:4, :] + acc[4:8, :]
    acc = acc[0:2, :] + acc[2:4, :]
    row = acc[0:1, :] + acc[1:2, :]
    row = jnp.where(mask_ref[0] > 0, _MASKED, row)
    out_ref[...] = row.reshape(1, 1, row.shape[-1])


def _topk_body(keep, scores_ref, idx_ref, msk_ref):
    s = scores_ref[...]
    B, N = s.shape
    col = lax.broadcasted_iota(jnp.int32, (B, N), 1)
    lane = lax.broadcasted_iota(jnp.int32, (B, keep), 1)
    rowbase = lax.broadcasted_iota(jnp.int32, (B, keep), 0) * N

    def body(j, carry):
        s, idxs, msks = carry
        m = jnp.max(s, axis=1, keepdims=True)
        pick = jnp.min(jnp.where(s == m, col, N), axis=1,
                       keepdims=True)
        sel = lane == j
        idxs = jnp.where(sel, pick, idxs)
        msks = jnp.where(sel, (m <= -1e38).astype(jnp.int32), msks)
        s = jnp.where(col == pick, -jnp.inf, s)
        return s, idxs, msks

    zero = jnp.zeros((B, keep), jnp.int32)
    _, idxs, msks = lax.fori_loop(0, keep, body, (s, zero, zero))
    idx_ref[...] = idxs + rowbase
    msk_ref[...] = msks


def _gather_rows(table, fidx):
    rows, d = table.shape
    g = fidx.shape[0]
    info = plsc.get_sparse_core_info()
    nw = info.num_cores * info.num_subcores
    bpw = g // nw
    mesh = plsc.VectorSubcoreMesh(core_axis_name="c", subcore_axis_name="s")

    @functools.partial(
        pl.kernel,
        out_type=jax.ShapeDtypeStruct((g, d), jnp.float32),
        mesh=mesh,
        scratch_types=[
            pltpu.VMEM((bpw,), jnp.int32),
            pltpu.VMEM((bpw, d), jnp.float32),
            pltpu.SemaphoreType.DMA,
        ],
    )
    def k(table_hbm, idx_hbm, out_hbm, idx_v, rows_v, sem):
        wid = lax.axis_index("s") * info.num_cores + lax.axis_index("c")
        base = wid * bpw
        pltpu.sync_copy(idx_hbm.at[pl.ds(base, bpw)], idx_v)
        pltpu.async_copy(table_hbm.at[idx_v], rows_v, sem).wait()
        pltpu.sync_copy(rows_v, out_hbm.at[pl.ds(base, bpw)])

    return k(table, fidx)


def kernel(query, blocks, block_mask, top_k, Wqd, bqd, Wqu, bqu, Wk, bk, Wh,
           bh, qn_w, kn_w):
    B, N, D = blocks.shape
    H = Wh.shape[0]
    I = Wqd.shape[0]
    keep = min(64, N)
    nt = N // _TILE_N

    qh, hw = pl.pallas_call(
        _proj_body,
        out_shape=(
            jax.ShapeDtypeStruct((B, H * I), jnp.float32),
            jax.ShapeDtypeStruct((B, H), jnp.float32),
        ),
    )(query, Wqd, bqd.reshape(1, I), Wqu, bqu.reshape(1, H * I), Wh,
      bh.reshape(1, H), qn_w.reshape(1, I))

    scores = pl.pallas_call(
        _score_body,
        grid=(B, nt),
        in_specs=[
            pl.BlockSpec((1, H, I), lambda b, n: (b, 0, 0)),
            pl.BlockSpec((1, H, 1), lambda b, n: (b, 0, 0)),
            pl.BlockSpec((I, D), lambda b, n: (0, 0)),
            pl.BlockSpec((I, 1), lambda b, n: (0, 0)),
            pl.BlockSpec((I, 1), lambda b, n: (0, 0)),
            pl.BlockSpec((1, _TILE_N, D), lambda b, n: (b, n, 0)),
            pl.BlockSpec((1, 1, _TILE_N), lambda b, n: (b, 0, n)),
        ],
        out_specs=pl.BlockSpec((1, 1, _TILE_N), lambda b, n: (b, 0, n)),
        out_shape=jax.ShapeDtypeStruct((B, 1, N), jnp.float32),
    )(qh.reshape(B, H, I), hw.reshape(B, H, 1), Wk, bk.reshape(I, 1),
      kn_w.reshape(I, 1), blocks, block_mask.reshape(B, 1, N).astype(jnp.int32))

    fidx, msk = pl.pallas_call(
        functools.partial(_topk_body, keep),
        out_shape=(
            jax.ShapeDtypeStruct((B, keep), jnp.int32),
            jax.ShapeDtypeStruct((B, keep), jnp.int32),
        ),
    )(scores.reshape(B, N))

    out = _gather_rows(blocks.reshape(B * N, D), fidx.reshape(B * keep))
    return out.reshape(B, keep, D), msk.astype(bool)


# speedup vs baseline: 1.0249x; 1.0249x over previous
import functools

import jax
import jax.numpy as jnp
from jax import lax
from jax.experimental import pallas as pl
from jax.experimental.pallas import tpu as pltpu
from jax.experimental.pallas import tpu_sc as plsc

_EPS = 1e-6
_TILE_N = 512
_MASKED = float(jnp.finfo(jnp.float32).min)
_CD11 = (((1,), (1,)), ((), ()))
_CD10 = (((1,), (0,)), ((), ()))


def _proj_body(q_ref, wqd_ref, bqd_ref, wqu_ref, bqu_ref, wh_ref, bh_ref,
               qnw_ref, qh_ref, hw_ref):
    q = q_ref[...]
    ql = lax.dot_general(q, wqd_ref[...], _CD11,
                         preferred_element_type=jnp.float32) + bqd_ref[...]
    ql = ql * lax.rsqrt(jnp.mean(ql * ql, axis=-1, keepdims=True) + _EPS)
    ql = ql * qnw_ref[...]
    qh_ref[...] = lax.dot_general(ql, wqu_ref[...], _CD11,
                                  preferred_element_type=jnp.float32) + bqu_ref[...]
    logits = lax.dot_general(q, wh_ref[...], _CD11,
                             preferred_element_type=jnp.float32) + bh_ref[...]
    m = jnp.max(logits, axis=-1, keepdims=True)
    e = jnp.exp(logits - m)
    hw_ref[...] = e / jnp.sum(e, axis=-1, keepdims=True)


def _score_topk_body(keep, qh_ref, hw_ref, wk_ref, bk_ref, knw_ref,
                     blocks_ref, mask_ref, idx_ref, msk_ref, scores_s):
    b = pl.program_id(0)
    n = pl.program_id(1)
    x = blocks_ref[0]
    tile = x.shape[0]
    kr = lax.dot_general(wk_ref[...], x, _CD11,
                         preferred_element_type=jnp.float32)
    krb = kr + bk_ref[...]
    sq = krb * krb
    part = sq[0:8, :]
    for i in range(1, 8):
        part = part + sq[8 * i:8 * i + 8, :]
    part = part[0:4, :] + part[4:8, :]
    part = part[0:2, :] + part[2:4, :]
    ms = (part[0:1, :] + part[1:2, :]) * (1.0 / 64.0)
    rs = lax.rsqrt(ms + _EPS)
    keys = (krb * rs) * knw_ref[...]
    sh = lax.dot_general(qh_ref[0].astype(jnp.bfloat16),
                         keys.astype(jnp.bfloat16), _CD10,
                         preferred_element_type=jnp.float32)
    prod = jnp.maximum(sh, 0.0) * hw_ref[0]
    acc = prod[0:8, :] + prod[8:16, :]
    acc = acc[0:4, :] + acc[4:8, :]
    acc = acc[0:2, :] + acc[2:4, :]
    row = acc[0:1, :] + acc[1:2, :]
    row = jnp.where(mask_ref[0] > 0, _MASKED, row)
    scores_s[pl.ds(b, 1), pl.ds(n * tile, tile)] = row

    @pl.when(jnp.logical_and(b == pl.num_programs(0) - 1,
                             n == pl.num_programs(1) - 1))
    def _topk():
        s = scores_s[...]
        B, N = s.shape
        col = lax.broadcasted_iota(jnp.int32, (B, N), 1)
        lane = lax.broadcasted_iota(jnp.int32, (B, keep), 1)
        rowbase = lax.broadcasted_iota(jnp.int32, (B, keep), 0) * N

        def body(j, carry):
            s, idxs, msks = carry
            m = jnp.max(s, axis=1, keepdims=True)
            pick = jnp.min(jnp.where(s == m, col, N), axis=1,
                           keepdims=True)
            sel = lane == j
            idxs = jnp.where(sel, pick, idxs)
            msks = jnp.where(sel, (m <= -1e38).astype(jnp.int32), msks)
            s = jnp.where(col == pick, -jnp.inf, s)
            return s, idxs, msks

        zero = jnp.zeros((B, keep), jnp.int32)
        _, idxs, msks = lax.fori_loop(0, keep, body, (s, zero, zero))
        idx_ref[...] = idxs + rowbase
        msk_ref[...] = msks


def _gather_rows(table, fidx):
    rows, d = table.shape
    g = fidx.shape[0]
    info = plsc.get_sparse_core_info()
    nw = info.num_cores * info.num_subcores
    bpw = g // nw
    mesh = plsc.VectorSubcoreMesh(core_axis_name="c", subcore_axis_name="s")

    @functools.partial(
        pl.kernel,
        out_type=jax.ShapeDtypeStruct((g, d), jnp.float32),
        mesh=mesh,
        scratch_types=[
            pltpu.VMEM((bpw,), jnp.int32),
            pltpu.VMEM((bpw, d), jnp.float32),
            pltpu.SemaphoreType.DMA,
        ],
    )
    def k(table_hbm, idx_hbm, out_hbm, idx_v, rows_v, sem):
        wid = lax.axis_index("s") * info.num_cores + lax.axis_index("c")
        base = wid * bpw
        pltpu.sync_copy(idx_hbm.at[pl.ds(base, bpw)], idx_v)
        pltpu.async_copy(table_hbm.at[idx_v], rows_v, sem).wait()
        pltpu.sync_copy(rows_v, out_hbm.at[pl.ds(base, bpw)])

    return k(table, fidx)


def kernel(query, blocks, block_mask, top_k, Wqd, bqd, Wqu, bqu, Wk, bk, Wh,
           bh, qn_w, kn_w):
    B, N, D = blocks.shape
    H = Wh.shape[0]
    I = Wqd.shape[0]
    keep = min(64, N)
    nt = N // _TILE_N

    qh, hw = pl.pallas_call(
        _proj_body,
        out_shape=(
            jax.ShapeDtypeStruct((B, H * I), jnp.float32),
            jax.ShapeDtypeStruct((B, H), jnp.float32),
        ),
    )(query, Wqd, bqd.reshape(1, I), Wqu, bqu.reshape(1, H * I), Wh,
      bh.reshape(1, H), qn_w.reshape(1, I))

    full = lambda b, n: (0, 0)
    fidx, msk = pl.pallas_call(
        functools.partial(_score_topk_body, keep),
        grid=(B, nt),
        in_specs=[
            pl.BlockSpec((1, H, I), lambda b, n: (b, 0, 0)),
            pl.BlockSpec((1, H, 1), lambda b, n: (b, 0, 0)),
            pl.BlockSpec((I, D), full),
            pl.BlockSpec((I, 1), full),
            pl.BlockSpec((I, 1), full),
            pl.BlockSpec((1, _TILE_N, D), lambda b, n: (b, n, 0)),
            pl.BlockSpec((1, 1, _TILE_N), lambda b, n: (b, 0, n)),
        ],
        out_specs=[pl.BlockSpec((B, keep), full),
                   pl.BlockSpec((B, keep), full)],
        out_shape=[jax.ShapeDtypeStruct((B, keep), jnp.int32),
                   jax.ShapeDtypeStruct((B, keep), jnp.int32)],
        scratch_shapes=[pltpu.VMEM((B, N), jnp.float32)],
    )(qh.reshape(B, H, I), hw.reshape(B, H, 1), Wk, bk.reshape(I, 1),
      kn_w.reshape(I, 1), blocks, block_mask.reshape(B, 1, N).astype(jnp.int32))

    out = _gather_rows(blocks.reshape(B * N, D), fidx.reshape(B * keep))
    return out.reshape(B, keep, D), msk.astype(bool)


# speedup vs baseline: 1.2838x; 1.2526x over previous
import functools

import jax
import jax.numpy as jnp
from jax import lax
from jax.experimental import pallas as pl
from jax.experimental.pallas import tpu as pltpu
from jax.experimental.pallas import tpu_sc as plsc

_EPS = 1e-6
_TILE_N = 1024
_MASKED = float(jnp.finfo(jnp.float32).min)
_CD11 = (((1,), (1,)), ((), ()))
_CD10 = (((1,), (0,)), ((), ()))


def _proj_body(q_ref, wqd_ref, bqd_ref, wqu_ref, bqu_ref, wh_ref, bh_ref,
               qnw_ref, qh_ref, hw_ref):
    q = q_ref[...]
    ql = lax.dot_general(q, wqd_ref[...], _CD11,
                         preferred_element_type=jnp.float32) + bqd_ref[...]
    ql = ql * lax.rsqrt(jnp.mean(ql * ql, axis=-1, keepdims=True) + _EPS)
    ql = ql * qnw_ref[...]
    qh_ref[...] = lax.dot_general(ql, wqu_ref[...], _CD11,
                                  preferred_element_type=jnp.float32) + bqu_ref[...]
    logits = lax.dot_general(q, wh_ref[...], _CD11,
                             preferred_element_type=jnp.float32) + bh_ref[...]
    m = jnp.max(logits, axis=-1, keepdims=True)
    e = jnp.exp(logits - m)
    hw_ref[...] = e / jnp.sum(e, axis=-1, keepdims=True)


def _score_topk_body(keep, qh_ref, hw_ref, wk_ref, bk_ref, knw_ref,
                     blocks_ref, mask_ref, idx_ref, msk_ref, scores_s):
    b = pl.program_id(0)
    n = pl.program_id(1)
    x = blocks_ref[0]
    tile = x.shape[0]
    kr = lax.dot_general(wk_ref[...], x, _CD11,
                         preferred_element_type=jnp.float32)
    krb = kr + bk_ref[...]
    sq = krb * krb
    part = sq[0:8, :]
    for i in range(1, 8):
        part = part + sq[8 * i:8 * i + 8, :]
    part = part[0:4, :] + part[4:8, :]
    part = part[0:2, :] + part[2:4, :]
    ms = (part[0:1, :] + part[1:2, :]) * (1.0 / 64.0)
    rs = lax.rsqrt(ms + _EPS)
    keys = (krb * rs) * knw_ref[...]
    sh = lax.dot_general(qh_ref[0].astype(jnp.bfloat16),
                         keys.astype(jnp.bfloat16), _CD10,
                         preferred_element_type=jnp.float32)
    prod = jnp.maximum(sh, 0.0) * hw_ref[0]
    acc = prod[0:8, :] + prod[8:16, :]
    acc = acc[0:4, :] + acc[4:8, :]
    acc = acc[0:2, :] + acc[2:4, :]
    row = acc[0:1, :] + acc[1:2, :]
    row = jnp.where(mask_ref[0] > 0, _MASKED, row)
    scores_s[pl.ds(b, 1), pl.ds(n * tile, tile)] = row

    @pl.when(jnp.logical_and(b == pl.num_programs(0) - 1,
                             n == pl.num_programs(1) - 1))
    def _topk():
        s = scores_s[...]
        B, N = s.shape
        col = lax.broadcasted_iota(jnp.int32, (B, N), 1)
        lane = lax.broadcasted_iota(jnp.int32, (B, keep), 1)
        rowbase = lax.broadcasted_iota(jnp.int32, (B, keep), 0) * N

        def body(j, carry):
            s, idxs, msks = carry
            m = jnp.max(s, axis=1, keepdims=True)
            pick = jnp.min(jnp.where(s == m, col, N), axis=1,
                           keepdims=True)
            sel = lane == j
            idxs = jnp.where(sel, pick, idxs)
            msks = jnp.where(sel, (m <= -1e38).astype(jnp.int32), msks)
            s = jnp.where(col == pick, -jnp.inf, s)
            return s, idxs, msks

        zero = jnp.zeros((B, keep), jnp.int32)
        _, idxs, msks = lax.fori_loop(0, keep, body, (s, zero, zero))
        idx_ref[...] = idxs + rowbase
        msk_ref[...] = msks


def _gather_rows(table, fidx):
    rows, d = table.shape
    g = fidx.shape[0]
    info = plsc.get_sparse_core_info()
    nw = info.num_cores * info.num_subcores
    bpw = g // nw
    mesh = plsc.VectorSubcoreMesh(core_axis_name="c", subcore_axis_name="s")

    @functools.partial(
        pl.kernel,
        out_type=jax.ShapeDtypeStruct((g, d), jnp.float32),
        mesh=mesh,
        scratch_types=[
            pltpu.VMEM((bpw,), jnp.int32),
            pltpu.VMEM((bpw, d), jnp.float32),
            pltpu.SemaphoreType.DMA,
        ],
    )
    def k(table_hbm, idx_hbm, out_hbm, idx_v, rows_v, sem):
        wid = lax.axis_index("s") * info.num_cores + lax.axis_index("c")
        base = wid * bpw
        pltpu.sync_copy(idx_hbm.at[pl.ds(base, bpw)], idx_v)
        pltpu.async_copy(table_hbm.at[idx_v], rows_v, sem).wait()
        pltpu.sync_copy(rows_v, out_hbm.at[pl.ds(base, bpw)])

    return k(table, fidx)


def kernel(query, blocks, block_mask, top_k, Wqd, bqd, Wqu, bqu, Wk, bk, Wh,
           bh, qn_w, kn_w):
    B, N, D = blocks.shape
    H = Wh.shape[0]
    I = Wqd.shape[0]
    keep = min(64, N)
    nt = N // _TILE_N

    qh, hw = pl.pallas_call(
        _proj_body,
        out_shape=(
            jax.ShapeDtypeStruct((B, H * I), jnp.float32),
            jax.ShapeDtypeStruct((B, H), jnp.float32),
        ),
    )(query, Wqd, bqd.reshape(1, I), Wqu, bqu.reshape(1, H * I), Wh,
      bh.reshape(1, H), qn_w.reshape(1, I))

    full = lambda b, n: (0, 0)
    fidx, msk = pl.pallas_call(
        functools.partial(_score_topk_body, keep),
        grid=(B, nt),
        in_specs=[
            pl.BlockSpec((1, H, I), lambda b, n: (b, 0, 0)),
            pl.BlockSpec((1, H, 1), lambda b, n: (b, 0, 0)),
            pl.BlockSpec((I, D), full),
            pl.BlockSpec((I, 1), full),
            pl.BlockSpec((I, 1), full),
            pl.BlockSpec((1, _TILE_N, D), lambda b, n: (b, n, 0)),
            pl.BlockSpec((1, 1, _TILE_N), lambda b, n: (b, 0, n)),
        ],
        out_specs=[pl.BlockSpec((B, keep), full),
                   pl.BlockSpec((B, keep), full)],
        out_shape=[jax.ShapeDtypeStruct((B, keep), jnp.int32),
                   jax.ShapeDtypeStruct((B, keep), jnp.int32)],
        scratch_shapes=[pltpu.VMEM((B, N), jnp.float32)],
    )(qh.reshape(B, H, I), hw.reshape(B, H, 1), Wk, bk.reshape(I, 1),
      kn_w.reshape(I, 1), blocks, block_mask.reshape(B, 1, N).astype(jnp.int32))

    out = _gather_rows(blocks.reshape(B * N, D), fidx.reshape(B * keep))
    return out.reshape(B, keep, D), msk.astype(bool)


# speedup vs baseline: 1.4697x; 1.1448x over previous
import functools

import jax
import jax.numpy as jnp
from jax import lax
from jax.experimental import pallas as pl
from jax.experimental.pallas import tpu as pltpu
from jax.experimental.pallas import tpu_sc as plsc

_EPS = 1e-6
_TILE_N = 2048
_MASKED = float(jnp.finfo(jnp.float32).min)
_CD11 = (((1,), (1,)), ((), ()))
_CD10 = (((1,), (0,)), ((), ()))


def _proj_body(q_ref, wqd_ref, bqd_ref, wqu_ref, bqu_ref, wh_ref, bh_ref,
               qnw_ref, qh_ref, hw_ref):
    q = q_ref[...]
    ql = lax.dot_general(q, wqd_ref[...], _CD11,
                         preferred_element_type=jnp.float32) + bqd_ref[...]
    ql = ql * lax.rsqrt(jnp.mean(ql * ql, axis=-1, keepdims=True) + _EPS)
    ql = ql * qnw_ref[...]
    qh_ref[...] = lax.dot_general(ql, wqu_ref[...], _CD11,
                                  preferred_element_type=jnp.float32) + bqu_ref[...]
    logits = lax.dot_general(q, wh_ref[...], _CD11,
                             preferred_element_type=jnp.float32) + bh_ref[...]
    m = jnp.max(logits, axis=-1, keepdims=True)
    e = jnp.exp(logits - m)
    hw_ref[...] = e / jnp.sum(e, axis=-1, keepdims=True)


def _score_topk_body(keep, qh_ref, hw_ref, wk_ref, bk_ref, knw_ref,
                     blocks_ref, mask_ref, idx_ref, msk_ref, scores_s):
    b = pl.program_id(0)
    n = pl.program_id(1)
    x = blocks_ref[0]
    tile = x.shape[0]
    kr = lax.dot_general(wk_ref[...], x, _CD11,
                         preferred_element_type=jnp.float32)
    krb = kr + bk_ref[...]
    sq = krb * krb
    part = sq[0:8, :]
    for i in range(1, 8):
        part = part + sq[8 * i:8 * i + 8, :]
    part = part[0:4, :] + part[4:8, :]
    part = part[0:2, :] + part[2:4, :]
    ms = (part[0:1, :] + part[1:2, :]) * (1.0 / 64.0)
    rs = lax.rsqrt(ms + _EPS)
    keys = (krb * rs) * knw_ref[...]
    sh = lax.dot_general(qh_ref[0].astype(jnp.bfloat16),
                         keys.astype(jnp.bfloat16), _CD10,
                         preferred_element_type=jnp.float32)
    prod = jnp.maximum(sh, 0.0) * hw_ref[0]
    acc = prod[0:8, :] + prod[8:16, :]
    acc = acc[0:4, :] + acc[4:8, :]
    acc = acc[0:2, :] + acc[2:4, :]
    row = acc[0:1, :] + acc[1:2, :]
    row = jnp.where(mask_ref[0] > 0, _MASKED, row)
    scores_s[pl.ds(b, 1), pl.ds(n * tile, tile)] = row

    @pl.when(jnp.logical_and(b == pl.num_programs(0) - 1,
                             n == pl.num_programs(1) - 1))
    def _topk():
        s = scores_s[...]
        B, N = s.shape
        col = lax.broadcasted_iota(jnp.int32, (B, N), 1)
        lane = lax.broadcasted_iota(jnp.int32, (B, keep), 1)
        rowbase = lax.broadcasted_iota(jnp.int32, (B, keep), 0) * N

        def body(j, carry):
            s, idxs, msks = carry
            m = jnp.max(s, axis=1, keepdims=True)
            pick = jnp.min(jnp.where(s == m, col, N), axis=1,
                           keepdims=True)
            sel = lane == j
            idxs = jnp.where(sel, pick, idxs)
            msks = jnp.where(sel, (m <= -1e38).astype(jnp.int32), msks)
            s = jnp.where(col == pick, -jnp.inf, s)
            return s, idxs, msks

        zero = jnp.zeros((B, keep), jnp.int32)
        _, idxs, msks = lax.fori_loop(0, keep, body, (s, zero, zero))
        idx_ref[...] = idxs + rowbase
        msk_ref[...] = msks


def _gather_rows(table, fidx):
    rows, d = table.shape
    g = fidx.shape[0]
    info = plsc.get_sparse_core_info()
    nw = info.num_cores * info.num_subcores
    bpw = g // nw
    mesh = plsc.VectorSubcoreMesh(core_axis_name="c", subcore_axis_name="s")

    @functools.partial(
        pl.kernel,
        out_type=jax.ShapeDtypeStruct((g, d), jnp.float32),
        mesh=mesh,
        scratch_types=[
            pltpu.VMEM((bpw,), jnp.int32),
            pltpu.VMEM((bpw, d), jnp.float32),
            pltpu.SemaphoreType.DMA,
        ],
    )
    def k(table_hbm, idx_hbm, out_hbm, idx_v, rows_v, sem):
        wid = lax.axis_index("s") * info.num_cores + lax.axis_index("c")
        base = wid * bpw
        pltpu.sync_copy(idx_hbm.at[pl.ds(base, bpw)], idx_v)
        pltpu.async_copy(table_hbm.at[idx_v], rows_v, sem).wait()
        pltpu.sync_copy(rows_v, out_hbm.at[pl.ds(base, bpw)])

    return k(table, fidx)


def kernel(query, blocks, block_mask, top_k, Wqd, bqd, Wqu, bqu, Wk, bk, Wh,
           bh, qn_w, kn_w):
    B, N, D = blocks.shape
    H = Wh.shape[0]
    I = Wqd.shape[0]
    keep = min(64, N)
    nt = N // _TILE_N

    qh, hw = pl.pallas_call(
        _proj_body,
        out_shape=(
            jax.ShapeDtypeStruct((B, H * I), jnp.float32),
            jax.ShapeDtypeStruct((B, H), jnp.float32),
        ),
    )(query, Wqd, bqd.reshape(1, I), Wqu, bqu.reshape(1, H * I), Wh,
      bh.reshape(1, H), qn_w.reshape(1, I))

    full = lambda b, n: (0, 0)
    fidx, msk = pl.pallas_call(
        functools.partial(_score_topk_body, keep),
        grid=(B, nt),
        in_specs=[
            pl.BlockSpec((1, H, I), lambda b, n: (b, 0, 0)),
            pl.BlockSpec((1, H, 1), lambda b, n: (b, 0, 0)),
            pl.BlockSpec((I, D), full),
            pl.BlockSpec((I, 1), full),
            pl.BlockSpec((I, 1), full),
            pl.BlockSpec((1, _TILE_N, D), lambda b, n: (b, n, 0)),
            pl.BlockSpec((1, 1, _TILE_N), lambda b, n: (b, 0, n)),
        ],
        out_specs=[pl.BlockSpec((B, keep), full),
                   pl.BlockSpec((B, keep), full)],
        out_shape=[jax.ShapeDtypeStruct((B, keep), jnp.int32),
                   jax.ShapeDtypeStruct((B, keep), jnp.int32)],
        scratch_shapes=[pltpu.VMEM((B, N), jnp.float32)],
    )(qh.reshape(B, H, I), hw.reshape(B, H, 1), Wk, bk.reshape(I, 1),
      kn_w.reshape(I, 1), blocks, block_mask.reshape(B, 1, N).astype(jnp.int32))

    out = _gather_rows(blocks.reshape(B * N, D), fidx.reshape(B * keep))
    return out.reshape(B, keep, D), msk.astype(bool)


# speedup vs baseline: 1.4760x; 1.0043x over previous
import functools

import jax
import jax.numpy as jnp
from jax import lax
from jax.experimental import pallas as pl
from jax.experimental.pallas import tpu as pltpu
from jax.experimental.pallas import tpu_sc as plsc

_EPS = 1e-6
_TILE_N = 4096
_MASKED = float(jnp.finfo(jnp.float32).min)
_CD11 = (((1,), (1,)), ((), ()))
_CD10 = (((1,), (0,)), ((), ()))


def _proj_body(q_ref, wqd_ref, bqd_ref, wqu_ref, bqu_ref, wh_ref, bh_ref,
               qnw_ref, qh_ref, hw_ref):
    q = q_ref[...]
    ql = lax.dot_general(q, wqd_ref[...], _CD11,
                         preferred_element_type=jnp.float32) + bqd_ref[...]
    ql = ql * lax.rsqrt(jnp.mean(ql * ql, axis=-1, keepdims=True) + _EPS)
    ql = ql * qnw_ref[...]
    qh_ref[...] = lax.dot_general(ql, wqu_ref[...], _CD11,
                                  preferred_element_type=jnp.float32) + bqu_ref[...]
    logits = lax.dot_general(q, wh_ref[...], _CD11,
                             preferred_element_type=jnp.float32) + bh_ref[...]
    m = jnp.max(logits, axis=-1, keepdims=True)
    e = jnp.exp(logits - m)
    hw_ref[...] = e / jnp.sum(e, axis=-1, keepdims=True)


def _score_topk_body(keep, qh_ref, hw_ref, wk_ref, bk_ref, knw_ref,
                     blocks_ref, mask_ref, idx_ref, msk_ref, scores_s):
    b = pl.program_id(0)
    n = pl.program_id(1)
    x = blocks_ref[0]
    tile = x.shape[0]
    kr = lax.dot_general(wk_ref[...], x, _CD11,
                         preferred_element_type=jnp.float32)
    krb = kr + bk_ref[...]
    sq = krb * krb
    part = sq[0:8, :]
    for i in range(1, 8):
        part = part + sq[8 * i:8 * i + 8, :]
    part = part[0:4, :] + part[4:8, :]
    part = part[0:2, :] + part[2:4, :]
    ms = (part[0:1, :] + part[1:2, :]) * (1.0 / 64.0)
    rs = lax.rsqrt(ms + _EPS)
    keys = (krb * rs) * knw_ref[...]
    sh = lax.dot_general(qh_ref[0].astype(jnp.bfloat16),
                         keys.astype(jnp.bfloat16), _CD10,
                         preferred_element_type=jnp.float32)
    prod = jnp.maximum(sh, 0.0) * hw_ref[0]
    acc = prod[0:8, :] + prod[8:16, :]
    acc = acc[0:4, :] + acc[4:8, :]
    acc = acc[0:2, :] + acc[2:4, :]
    row = acc[0:1, :] + acc[1:2, :]
    row = jnp.where(mask_ref[0] > 0, _MASKED, row)
    scores_s[pl.ds(b, 1), pl.ds(n * tile, tile)] = row

    @pl.when(jnp.logical_and(b == pl.num_programs(0) - 1,
                             n == pl.num_programs(1) - 1))
    def _topk():
        s = scores_s[...]
        B, N = s.shape
        col = lax.broadcasted_iota(jnp.int32, (B, N), 1)
        lane = lax.broadcasted_iota(jnp.int32, (B, keep), 1)
        rowbase = lax.broadcasted_iota(jnp.int32, (B, keep), 0) * N

        def body(j, carry):
            s, idxs, msks = carry
            m = jnp.max(s, axis=1, keepdims=True)
            pick = jnp.min(jnp.where(s == m, col, N), axis=1,
                           keepdims=True)
            sel = lane == j
            idxs = jnp.where(sel, pick, idxs)
            msks = jnp.where(sel, (m <= -1e38).astype(jnp.int32), msks)
            s = jnp.where(col == pick, -jnp.inf, s)
            return s, idxs, msks

        zero = jnp.zeros((B, keep), jnp.int32)
        _, idxs, msks = lax.fori_loop(0, keep, body, (s, zero, zero))
        idx_ref[...] = idxs + rowbase
        msk_ref[...] = msks


def _gather_rows(table, fidx):
    rows, d = table.shape
    g = fidx.shape[0]
    info = plsc.get_sparse_core_info()
    nw = info.num_cores * info.num_subcores
    bpw = g // nw
    mesh = plsc.VectorSubcoreMesh(core_axis_name="c", subcore_axis_name="s")

    @functools.partial(
        pl.kernel,
        out_type=jax.ShapeDtypeStruct((g, d), jnp.float32),
        mesh=mesh,
        scratch_types=[
            pltpu.VMEM((bpw,), jnp.int32),
            pltpu.VMEM((bpw, d), jnp.float32),
            pltpu.SemaphoreType.DMA,
        ],
    )
    def k(table_hbm, idx_hbm, out_hbm, idx_v, rows_v, sem):
        wid = lax.axis_index("s") * info.num_cores + lax.axis_index("c")
        base = wid * bpw
        pltpu.sync_copy(idx_hbm.at[pl.ds(base, bpw)], idx_v)
        pltpu.async_copy(table_hbm.at[idx_v], rows_v, sem).wait()
        pltpu.sync_copy(rows_v, out_hbm.at[pl.ds(base, bpw)])

    return k(table, fidx)


def kernel(query, blocks, block_mask, top_k, Wqd, bqd, Wqu, bqu, Wk, bk, Wh,
           bh, qn_w, kn_w):
    B, N, D = blocks.shape
    H = Wh.shape[0]
    I = Wqd.shape[0]
    keep = min(64, N)
    nt = N // _TILE_N

    qh, hw = pl.pallas_call(
        _proj_body,
        out_shape=(
            jax.ShapeDtypeStruct((B, H * I), jnp.float32),
            jax.ShapeDtypeStruct((B, H), jnp.float32),
        ),
    )(query, Wqd, bqd.reshape(1, I), Wqu, bqu.reshape(1, H * I), Wh,
      bh.reshape(1, H), qn_w.reshape(1, I))

    full = lambda b, n: (0, 0)
    fidx, msk = pl.pallas_call(
        functools.partial(_score_topk_body, keep),
        grid=(B, nt),
        in_specs=[
            pl.BlockSpec((1, H, I), lambda b, n: (b, 0, 0)),
            pl.BlockSpec((1, H, 1), lambda b, n: (b, 0, 0)),
            pl.BlockSpec((I, D), full),
            pl.BlockSpec((I, 1), full),
            pl.BlockSpec((I, 1), full),
            pl.BlockSpec((1, _TILE_N, D), lambda b, n: (b, n, 0)),
            pl.BlockSpec((1, 1, _TILE_N), lambda b, n: (b, 0, n)),
        ],
        out_specs=[pl.BlockSpec((B, keep), full),
                   pl.BlockSpec((B, keep), full)],
        out_shape=[jax.ShapeDtypeStruct((B, keep), jnp.int32),
                   jax.ShapeDtypeStruct((B, keep), jnp.int32)],
        scratch_shapes=[pltpu.VMEM((B, N), jnp.float32)],
    )(qh.reshape(B, H, I), hw.reshape(B, H, 1), Wk, bk.reshape(I, 1),
      kn_w.reshape(I, 1), blocks, block_mask.reshape(B, 1, N).astype(jnp.int32))

    out = _gather_rows(blocks.reshape(B * N, D), fidx.reshape(B * keep))
    return out.reshape(B, keep, D), msk.astype(bool)


# speedup vs baseline: 1.5225x; 1.0315x over previous
import functools

import jax
import jax.numpy as jnp
from jax import lax
from jax.experimental import pallas as pl
from jax.experimental.pallas import tpu as pltpu
from jax.experimental.pallas import tpu_sc as plsc

_EPS = 1e-6
_TILE_N = 4096
_MASKED = float(jnp.finfo(jnp.float32).min)
_CD11 = (((1,), (1,)), ((), ()))
_CD10 = (((1,), (0,)), ((), ()))


def _proj_body(q_ref, wqd_ref, bqd_ref, wqu_ref, bqu_ref, wh_ref, bh_ref,
               qnw_ref, qh_ref, hw_ref):
    q = q_ref[...]
    ql = lax.dot_general(q, wqd_ref[...], _CD11,
                         preferred_element_type=jnp.float32) + bqd_ref[...]
    ql = ql * lax.rsqrt(jnp.mean(ql * ql, axis=-1, keepdims=True) + _EPS)
    ql = ql * qnw_ref[...]
    qh_ref[...] = lax.dot_general(ql, wqu_ref[...], _CD11,
                                  preferred_element_type=jnp.float32) + bqu_ref[...]
    logits = lax.dot_general(q, wh_ref[...], _CD11,
                             preferred_element_type=jnp.float32) + bh_ref[...]
    m = jnp.max(logits, axis=-1, keepdims=True)
    e = jnp.exp(logits - m)
    hw_ref[...] = e / jnp.sum(e, axis=-1, keepdims=True)


def _score_topk_body(keep, q_ref, wqd_ref, bqd_ref, wqu_ref, bqu_ref,
                     wh_ref, bh_ref, qnw_ref, wk_ref, bk_ref, knw_ref,
                     blocks_ref, mask_ref, idx_ref, msk_ref, scores_s,
                     qh3_s, hw3_s):
    b = pl.program_id(0)
    n = pl.program_id(1)

    @pl.when(jnp.logical_and(b == 0, n == 0))
    def _proj():
        q = q_ref[...]
        B, D = q.shape
        H, I = qh3_s.shape[1], qh3_s.shape[2]
        ql = lax.dot_general(q, wqd_ref[...], _CD11,
                             preferred_element_type=jnp.float32) + bqd_ref[...]
        ql = ql * lax.rsqrt(jnp.mean(ql * ql, axis=-1, keepdims=True) + _EPS)
        ql = ql * qnw_ref[...]
        qh = lax.dot_general(ql, wqu_ref[...], _CD11,
                             preferred_element_type=jnp.float32) + bqu_ref[...]
        qh3_s[...] = qh.reshape(B, H, I)
        logits = lax.dot_general(q, wh_ref[...], _CD11,
                                 preferred_element_type=jnp.float32) + bh_ref[...]
        m = jnp.max(logits, axis=-1, keepdims=True)
        e = jnp.exp(logits - m)
        hw3_s[...] = (e / jnp.sum(e, axis=-1, keepdims=True)).reshape(B, H, 1)

    x = blocks_ref[0]
    tile = x.shape[0]
    kr = lax.dot_general(wk_ref[...], x, _CD11,
                         preferred_element_type=jnp.float32)
    krb = kr + bk_ref[...]
    sq = krb * krb
    part = sq[0:8, :]
    for i in range(1, 8):
        part = part + sq[8 * i:8 * i + 8, :]
    part = part[0:4, :] + part[4:8, :]
    part = part[0:2, :] + part[2:4, :]
    ms = (part[0:1, :] + part[1:2, :]) * (1.0 / 64.0)
    rs = lax.rsqrt(ms + _EPS)
    keys = (krb * rs) * knw_ref[...]
    sh = lax.dot_general(qh3_s[b].astype(jnp.bfloat16),
                         keys.astype(jnp.bfloat16), _CD10,
                         preferred_element_type=jnp.float32)
    prod = jnp.maximum(sh, 0.0) * hw3_s[b]
    acc = prod[0:8, :] + prod[8:16, :]
    acc = acc[0:4, :] + acc[4:8, :]
    acc = acc[0:2, :] + acc[2:4, :]
    row = acc[0:1, :] + acc[1:2, :]
    row = jnp.where(mask_ref[0] > 0, _MASKED, row)
    scores_s[pl.ds(b, 1), pl.ds(n * tile, tile)] = row

    @pl.when(jnp.logical_and(b == pl.num_programs(0) - 1,
                             n == pl.num_programs(1) - 1))
    def _topk():
        s = scores_s[...]
        B, N = s.shape
        col = lax.broadcasted_iota(jnp.int32, (B, N), 1)
        lane = lax.broadcasted_iota(jnp.int32, (B, keep), 1)
        rowbase = lax.broadcasted_iota(jnp.int32, (B, keep), 0) * N

        def body(j, carry):
            s, idxs, msks = carry
            m = jnp.max(s, axis=1, keepdims=True)
            pick = jnp.min(jnp.where(s == m, col, N), axis=1,
                           keepdims=True)
            sel = lane == j
            idxs = jnp.where(sel, pick, idxs)
            msks = jnp.where(sel, (m <= -1e38).astype(jnp.int32), msks)
            s = jnp.where(col == pick, -jnp.inf, s)
            return s, idxs, msks

        zero = jnp.zeros((B, keep), jnp.int32)
        _, idxs, msks = lax.fori_loop(0, keep, body, (s, zero, zero))
        idx_ref[...] = idxs + rowbase
        msk_ref[...] = msks


def _gather_rows(table, fidx):
    rows, d = table.shape
    g = fidx.shape[0]
    info = plsc.get_sparse_core_info()
    nw = info.num_cores * info.num_subcores
    bpw = g // nw
    mesh = plsc.VectorSubcoreMesh(core_axis_name="c", subcore_axis_name="s")

    @functools.partial(
        pl.kernel,
        out_type=jax.ShapeDtypeStruct((g, d), jnp.float32),
        mesh=mesh,
        scratch_types=[
            pltpu.VMEM((bpw,), jnp.int32),
            pltpu.VMEM((bpw, d), jnp.float32),
            pltpu.SemaphoreType.DMA,
        ],
    )
    def k(table_hbm, idx_hbm, out_hbm, idx_v, rows_v, sem):
        wid = lax.axis_index("s") * info.num_cores + lax.axis_index("c")
        base = wid * bpw
        pltpu.sync_copy(idx_hbm.at[pl.ds(base, bpw)], idx_v)
        pltpu.async_copy(table_hbm.at[idx_v], rows_v, sem).wait()
        pltpu.sync_copy(rows_v, out_hbm.at[pl.ds(base, bpw)])

    return k(table, fidx)


def kernel(query, blocks, block_mask, top_k, Wqd, bqd, Wqu, bqu, Wk, bk, Wh,
           bh, qn_w, kn_w):
    B, N, D = blocks.shape
    H = Wh.shape[0]
    I = Wqd.shape[0]
    keep = min(64, N)
    nt = N // _TILE_N

    full = lambda b, n: (0, 0)
    fidx, msk = pl.pallas_call(
        functools.partial(_score_topk_body, keep),
        grid=(B, nt),
        in_specs=[
            pl.BlockSpec((B, D), full),
            pl.BlockSpec((I, D), full),
            pl.BlockSpec((1, I), full),
            pl.BlockSpec((H * I, I), full),
            pl.BlockSpec((1, H * I), full),
            pl.BlockSpec((H, D), full),
            pl.BlockSpec((1, H), full),
            pl.BlockSpec((1, I), full),
            pl.BlockSpec((I, D), full),
            pl.BlockSpec((I, 1), full),
            pl.BlockSpec((I, 1), full),
            pl.BlockSpec((1, _TILE_N, D), lambda b, n: (b, n, 0)),
            pl.BlockSpec((1, 1, _TILE_N), lambda b, n: (b, 0, n)),
        ],
        out_specs=[pl.BlockSpec((B, keep), full),
                   pl.BlockSpec((B, keep), full)],
        out_shape=[jax.ShapeDtypeStruct((B, keep), jnp.int32),
                   jax.ShapeDtypeStruct((B, keep), jnp.int32)],
        scratch_shapes=[pltpu.VMEM((B, N), jnp.float32),
                        pltpu.VMEM((B, H, I), jnp.float32),
                        pltpu.VMEM((B, H, 1), jnp.float32)],
    )(query, Wqd, bqd.reshape(1, I), Wqu, bqu.reshape(1, H * I), Wh,
      bh.reshape(1, H), qn_w.reshape(1, I), Wk, bk.reshape(I, 1),
      kn_w.reshape(I, 1), blocks, block_mask.reshape(B, 1, N).astype(jnp.int32))

    out = _gather_rows(blocks.reshape(B * N, D), fidx.reshape(B * keep))
    return out.reshape(B, keep, D), msk.astype(bool)
